# Initial kernel scaffold; baseline (speedup 1.0000x reference)
import jax
import jax.numpy as jnp
from jax.experimental import pallas as pl


def kernel(b, A, params, edge_index):
    raise NotImplementedError("write your pallas kernel here")


# speedup vs baseline: 4.3339x; 4.3339x over previous
import functools

import jax
import jax.numpy as jnp
from jax.experimental import pallas as pl
from jax.experimental.pallas import tpu as pltpu

P = 50
M = 256
N = 512
H = 128
B = 16
K = 3
E = 400
ALPHA_MAX = 0.01
TAU_MAX = 0.01
RHO_MAX = 0.1
ETA_MAX = 0.01

PCHUNK = 8
NPGRID = (P + PCHUNK - 1) // PCHUNK
DCHUNK = 3200
NDGRID = (P * 4 * H) // DCHUNK


def _lrelu(x):
    return jnp.where(x >= 0, x, 0.01 * x)


def _edge_kernel(ei_ref, ghat_ref, lap_ref, deg_ref):
    src = ei_ref[0:1, :]
    dst = ei_ref[1:2, :]
    rows = jax.lax.broadcasted_iota(jnp.int32, (P, E), 0)
    s_oh = (rows == src).astype(jnp.float32)
    d_oh = (rows == dst).astype(jnp.float32)
    deg = jnp.sum(s_oh, axis=1, keepdims=True)
    dself = deg + 1.0
    inv_dself = 1.0 / dself
    ds_src = jnp.sum(s_oh * dself, axis=0, keepdims=True)
    ds_dst = jnp.sum(d_oh * dself, axis=0, keepdims=True)
    norm = jax.lax.rsqrt(ds_src * ds_dst)
    ghat = jax.lax.dot_general(
        d_oh * norm, s_oh, (((1,), (1,)), ((), ())),
        preferred_element_type=jnp.float32)
    eye = (jax.lax.broadcasted_iota(jnp.int32, (P, P), 0)
           == jax.lax.broadcasted_iota(jnp.int32, (P, P), 1))
    ghat_ref[...] = ghat + eye.astype(jnp.float32) * inv_dself
    dif = s_oh - d_oh
    lap_ref[...] = jax.lax.dot_general(
        dif, dif, (((1,), (1,)), ((), ())),
        preferred_element_type=jnp.float32)
    deg_ref[...] = jnp.broadcast_to(deg, (P, 8))


def _atb_kernel(b_ref, a_ref, out_ref):
    for i in range(PCHUNK):
        out_ref[:, i, :] = jnp.dot(b_ref[:, i, :], a_ref[i],
                                   preferred_element_type=jnp.float32)


def _atay_kernel(y_ref, a_ref, out_ref):
    for i in range(PCHUNK):
        a = a_ref[i]
        u = jax.lax.dot_general(y_ref[:, i, :], a, (((1,), (1,)), ((), ())),
                                preferred_element_type=jnp.float32)
        out_ref[:, i, :] = jnp.dot(u, a, preferred_element_type=jnp.float32)


def _gnn_kernel(ay_ref, atb_ref, g_ref,
                w1a_ref, w1b_ref, w2_ref, w3_ref, w4_ref, w5_ref,
                bc_ref, gg_ref, be_ref, lng_ref, lnb_ref, out_ref):
    ghat = g_ref[...]

    def bn(x, lo, fo):
        mu = jnp.mean(x, axis=0, keepdims=True)
        var = jnp.mean((x - mu) ** 2, axis=0, keepdims=True)
        g = gg_ref[0:1, lo:lo + fo]
        bb = be_ref[0:1, lo:lo + fo]
        return g * (x - mu) / jnp.sqrt(var + 1e-5) + bb

    def layer(xw, lo, fo):
        h = _lrelu(jnp.dot(ghat, xw, preferred_element_type=jnp.float32)
                   + bc_ref[0:1, lo:lo + fo])
        return bn(h, lo, fo)

    xw = (jnp.dot(ay_ref[0], w1a_ref[...], preferred_element_type=jnp.float32)
          + jnp.dot(atb_ref[0], w1b_ref[...], preferred_element_type=jnp.float32))
    h = layer(xw, 0, H)
    h = layer(jnp.dot(h, w2_ref[...], preferred_element_type=jnp.float32), H, 2 * H)
    h = layer(jnp.dot(h, w3_ref[...], preferred_element_type=jnp.float32), 3 * H, 4 * H)
    h = layer(jnp.dot(h, w4_ref[...], preferred_element_type=jnp.float32), 7 * H, 4 * H)
    h = layer(jnp.dot(h, w5_ref[...], preferred_element_type=jnp.float32), 11 * H, 4 * H)
    mu = jnp.mean(h, axis=1, keepdims=True)
    var = jnp.mean((h - mu) ** 2, axis=1, keepdims=True)
    out_ref[0] = (lng_ref[...] * (h - mu) / jnp.sqrt(var + 1e-5)
                  + lnb_ref[...])


def _dec_kernel(h_ref, d1_ref, db1_ref, dg1_ref, dbe1_ref,
                d2_ref, db2_ref, dg2_ref, dbe2_ref,
                d3_ref, db3_ref, dg3_ref, dbe3_ref,
                fc_ref, fcb_ref, hyp_ref, acc_ref):
    c = pl.program_id(0)

    @pl.when(c == 0)
    def _():
        acc_ref[...] = jnp.zeros_like(acc_ref)

    acc_ref[...] += jnp.dot(h_ref[...], d1_ref[...],
                            preferred_element_type=jnp.float32)

    @pl.when(c == NDGRID - 1)
    def _():
        def ln(x, g_r, b_r):
            mu = jnp.mean(x, axis=1, keepdims=True)
            var = jnp.mean((x - mu) ** 2, axis=1, keepdims=True)
            return g_r[...] * (x - mu) / jnp.sqrt(var + 1e-5) + b_r[...]

        h = _lrelu(ln(acc_ref[...] + db1_ref[...], dg1_ref, dbe1_ref))
        h = _lrelu(ln(jnp.dot(h, d2_ref[...], preferred_element_type=jnp.float32)
                      + db2_ref[...], dg2_ref, dbe2_ref))
        h = _lrelu(ln(jnp.dot(h, d3_ref[...], preferred_element_type=jnp.float32)
                      + db3_ref[...], dg3_ref, dbe3_ref))
        z = jnp.dot(h, fc_ref[...], preferred_element_type=jnp.float32) + fcb_ref[...]
        hyp_ref[...] = jnp.clip(jax.nn.sigmoid(z), 1e-4, 0.99)


def _update_kernel(y_ref, u_ref, dl_ref, ay_ref, atb_ref, hyp_ref, lap_ref,
                   deg_ref, yn_ref, dn_ref, un_ref, *, mg, mv):
    bi = pl.program_id(0)
    hrow = hyp_ref[pl.ds(bi, 1), :]
    alpha = hrow[:, 0:1] * ALPHA_MAX
    tau = hrow[:, 1:2] * TAU_MAX
    rho = hrow[:, 2:3] * RHO_MAX
    eta = hrow[:, 3:4] * ETA_MAX
    y = y_ref[0]
    grad = (ay_ref[0] - atb_ref[0] + jnp.sign(y) * tau
            + u_ref[0] * deg_ref[:, 0:1] + dl_ref[0] * rho)
    grad = jnp.clip(grad, -mg, mg)
    y_n = jnp.clip(y - alpha * grad, -mv, mv)
    d_n = jnp.dot(lap_ref[...], y_n, preferred_element_type=jnp.float32)
    yn_ref[0] = y_n
    dn_ref[0] = d_n
    un_ref[0] = jnp.clip(u_ref[0] + d_n * eta, -mv, mv)


def _full(shape):
    return pl.BlockSpec(shape, lambda *_: tuple(0 for _ in shape))


def kernel(b, A, params, edge_index):
    p = params
    f32 = jnp.float32
    A0 = A[0]
    bmat = b[..., 0]

    ghat, lap, deg = pl.pallas_call(
        _edge_kernel,
        out_shape=[jax.ShapeDtypeStruct((P, P), f32),
                   jax.ShapeDtypeStruct((P, P), f32),
                   jax.ShapeDtypeStruct((P, 8), f32)],
    )(edge_index)

    atb = pl.pallas_call(
        _atb_kernel,
        grid=(NPGRID,),
        in_specs=[pl.BlockSpec((B, PCHUNK, M), lambda j: (0, j, 0)),
                  pl.BlockSpec((PCHUNK, M, N), lambda j: (j, 0, 0))],
        out_specs=pl.BlockSpec((B, PCHUNK, N), lambda j: (0, j, 0)),
        out_shape=jax.ShapeDtypeStruct((B, P, N), f32),
    )(bmat, A0)

    atay = pl.pallas_call(
        _atay_kernel,
        grid=(NPGRID,),
        in_specs=[pl.BlockSpec((B, PCHUNK, N), lambda j: (0, j, 0)),
                  pl.BlockSpec((PCHUNK, M, N), lambda j: (j, 0, 0))],
        out_specs=pl.BlockSpec((B, PCHUNK, N), lambda j: (0, j, 0)),
        out_shape=jax.ShapeDtypeStruct((B, P, N), f32),
    )

    bc = jnp.concatenate([p['bc%d' % i] for i in range(1, 6)])[None, :]
    gg = jnp.concatenate([p['g%d' % i] for i in range(1, 6)])[None, :]
    be = jnp.concatenate([p['be%d' % i] for i in range(1, 6)])[None, :]
    w1a = p['W1'][:N]
    w1b = p['W1'][N:]

    gnn = pl.pallas_call(
        _gnn_kernel,
        grid=(B,),
        in_specs=[pl.BlockSpec((1, P, N), lambda i: (i, 0, 0)),
                  pl.BlockSpec((1, P, N), lambda i: (i, 0, 0)),
                  _full((P, P)),
                  _full((N, H)), _full((N, H)), _full((H, 2 * H)),
                  _full((2 * H, 4 * H)), _full((4 * H, 4 * H)),
                  _full((4 * H, 4 * H)),
                  _full((1, 15 * H)), _full((1, 15 * H)), _full((1, 15 * H)),
                  _full((1, 4 * H)), _full((1, 4 * H))],
        out_specs=pl.BlockSpec((1, P, 4 * H), lambda i: (i, 0, 0)),
        out_shape=jax.ShapeDtypeStruct((B, P, 4 * H), f32),
    )

    dec = pl.pallas_call(
        _dec_kernel,
        grid=(NDGRID,),
        in_specs=[pl.BlockSpec((B, DCHUNK), lambda c: (0, c)),
                  pl.BlockSpec((DCHUNK, 4 * H), lambda c: (c, 0)),
                  _full((1, 4 * H)), _full((1, 4 * H)), _full((1, 4 * H)),
                  _full((4 * H, 2 * H)),
                  _full((1, 2 * H)), _full((1, 2 * H)), _full((1, 2 * H)),
                  _full((2 * H, H)),
                  _full((1, H)), _full((1, H)), _full((1, H)),
                  _full((H, 4)), _full((1, 4))],
        out_specs=_full((B, 4)),
        out_shape=jax.ShapeDtypeStruct((B, 4), f32),
        scratch_shapes=[pltpu.VMEM((B, 4 * H), f32)],
    )
    dec_args = (p['db1'][None, :], p['dg1'][None, :], p['dbe1'][None, :],
                p['D2'], p['db2'][None, :], p['dg2'][None, :], p['dbe2'][None, :],
                p['D3'], p['db3'][None, :], p['dg3'][None, :], p['dbe3'][None, :],
                p['FC'], p['fcb'][None, :])

    def make_update(k):
        mg = max(30.0, 100.0 - k)
        mv = max(10.0, 200.0 - k * 3)
        blk = pl.BlockSpec((1, P, N), lambda i: (i, 0, 0))
        return pl.pallas_call(
            functools.partial(_update_kernel, mg=mg, mv=mv),
            grid=(B,),
            in_specs=[blk, blk, blk, blk, blk,
                      _full((B, 4)), _full((P, P)), _full((P, 8))],
            out_specs=[blk, blk, blk],
            out_shape=[jax.ShapeDtypeStruct((B, P, N), f32)] * 3,
        )

    zeros = jnp.zeros((B, P, N), f32)
    y_k, u_k, dl_k = zeros, zeros, zeros
    ys = []
    for k in range(K):
        ay = zeros if k == 0 else atay(y_k, A0)
        hnodes = gnn(ay, atb, ghat, w1a, w1b, p['W2'], p['W3'], p['W4'],
                     p['W5'], bc, gg, be, p['lng'][None, :], p['lnb'][None, :])
        hyp = dec(hnodes.reshape(B, P * 4 * H), p['D1'], *dec_args)
        y_k, dl_k, u_k = make_update(k)(y_k, u_k, dl_k, ay, atb, hyp, lap, deg)
        ys.append(y_k)
    return jnp.stack(ys)[..., None]


# speedup vs baseline: 6.3304x; 1.4607x over previous
import functools

import jax
import jax.numpy as jnp
from jax.experimental import pallas as pl
from jax.experimental.pallas import tpu as pltpu

P = 50
PP = 64
M = 256
N = 512
H = 128
B = 16
K = 3
E = 400
ALPHA_MAX = 0.01
TAU_MAX = 0.01
RHO_MAX = 0.1
ETA_MAX = 0.01

PCHUNK = 8
NPGRID = (P + PCHUNK - 1) // PCHUNK
DCHUNK = 3200
NDGRID = (P * 4 * H) // DCHUNK


def _lrelu(x):
    return jnp.where(x >= 0, x, 0.01 * x)


def _edge_kernel(ei_ref, ghat_ref, lap_ref, deg_ref):
    src = ei_ref[0:1, :]
    dst = ei_ref[1:2, :]
    rows = jax.lax.broadcasted_iota(jnp.int32, (PP, E), 0)
    s_oh = (rows == src).astype(jnp.float32)
    d_oh = (rows == dst).astype(jnp.float32)
    deg = jnp.sum(s_oh, axis=1, keepdims=True)
    dself = deg + 1.0
    ds_src = jnp.sum(s_oh * dself, axis=0, keepdims=True)
    ds_dst = jnp.sum(d_oh * dself, axis=0, keepdims=True)
    norm = jax.lax.rsqrt(ds_src * ds_dst)
    ghat = jax.lax.dot_general(
        d_oh * norm, s_oh, (((1,), (1,)), ((), ())),
        preferred_element_type=jnp.float32)
    ri = jax.lax.broadcasted_iota(jnp.int32, (PP, PP), 0)
    ci = jax.lax.broadcasted_iota(jnp.int32, (PP, PP), 1)
    eye = ((ri == ci) & (ri < P)).astype(jnp.float32)
    ghat_ref[...] = ghat + eye / dself
    dif = s_oh - d_oh
    lap_ref[...] = jax.lax.dot_general(
        dif, dif, (((1,), (1,)), ((), ())),
        preferred_element_type=jnp.float32)
    deg_ref[...] = jnp.broadcast_to(deg, (PP, 8))


def _atb_kernel(b_ref, a_ref, out_ref):
    for i in range(PCHUNK):
        out_ref[:, i, :] = jnp.dot(b_ref[:, i, :], a_ref[i],
                                   preferred_element_type=jnp.float32)


def _atay_kernel(y_ref, a_ref, out_ref):
    for i in range(PCHUNK):
        a = a_ref[i]
        u = jax.lax.dot_general(y_ref[:, i, :], a, (((1,), (1,)), ((), ())),
                                preferred_element_type=jnp.float32)
        out_ref[:, i, :] = jnp.dot(u, a, preferred_element_type=jnp.float32)


def _gnn_kernel(ay_ref, atb_ref, g_ref,
                w1a_ref, w1b_ref, w2_ref, w3_ref, w4_ref, w5_ref,
                bc_ref, gg_ref, be_ref, lng_ref, lnb_ref, out_ref):
    g_b = jnp.broadcast_to(g_ref[...], (B, PP, PP))
    mask = (jax.lax.broadcasted_iota(jnp.int32, (1, PP, 1), 1)
            < P).astype(jnp.float32)

    def layer(xw, lo, fo):
        x3 = xw.reshape(B, PP, fo)
        agg = jax.lax.dot_general(
            g_b, x3, (((2,), (1,)), ((0,), (0,))),
            preferred_element_type=jnp.float32)
        h = _lrelu(agg + bc_ref[0:1, lo:lo + fo].reshape(1, 1, fo))
        mu = jnp.sum(h * mask, axis=1, keepdims=True) / P
        var = jnp.sum(((h - mu) ** 2) * mask, axis=1, keepdims=True) / P
        h = (gg_ref[0:1, lo:lo + fo].reshape(1, 1, fo) * (h - mu)
             / jnp.sqrt(var + 1e-5)
             + be_ref[0:1, lo:lo + fo].reshape(1, 1, fo))
        return h.reshape(B * PP, fo)

    x2b = atb_ref[...].reshape(B * PP, N)
    xw = jnp.dot(x2b, w1b_ref[...], preferred_element_type=jnp.float32)
    if ay_ref is not None:
        x2a = ay_ref[...].reshape(B * PP, N)
        xw = xw + jnp.dot(x2a, w1a_ref[...], preferred_element_type=jnp.float32)
    h = layer(xw, 0, H)
    h = layer(jnp.dot(h, w2_ref[...], preferred_element_type=jnp.float32),
              H, 2 * H)
    h = layer(jnp.dot(h, w3_ref[...], preferred_element_type=jnp.float32),
              3 * H, 4 * H)
    h = layer(jnp.dot(h, w4_ref[...], preferred_element_type=jnp.float32),
              7 * H, 4 * H)
    h = layer(jnp.dot(h, w5_ref[...], preferred_element_type=jnp.float32),
              11 * H, 4 * H)
    h3 = h.reshape(B, PP, 4 * H)
    mu = jnp.mean(h3, axis=2, keepdims=True)
    var = jnp.mean((h3 - mu) ** 2, axis=2, keepdims=True)
    hln = (lng_ref[...].reshape(1, 1, 4 * H) * (h3 - mu)
           / jnp.sqrt(var + 1e-5) + lnb_ref[...].reshape(1, 1, 4 * H))
    out_ref[...] = hln[:, :P, :]


def _gnn_kernel_k0(atb_ref, g_ref, w1b_ref, w2_ref, w3_ref, w4_ref, w5_ref,
                   bc_ref, gg_ref, be_ref, lng_ref, lnb_ref, out_ref):
    _gnn_kernel(None, atb_ref, g_ref, None, w1b_ref, w2_ref, w3_ref, w4_ref,
                w5_ref, bc_ref, gg_ref, be_ref, lng_ref, lnb_ref, out_ref)


def _dec_kernel(h_ref, d1_ref, db1_ref, dg1_ref, dbe1_ref,
                d2_ref, db2_ref, dg2_ref, dbe2_ref,
                d3_ref, db3_ref, dg3_ref, dbe3_ref,
                fc_ref, fcb_ref, hyp_ref, acc_ref):
    c = pl.program_id(0)

    @pl.when(c == 0)
    def _():
        acc_ref[...] = jnp.zeros_like(acc_ref)

    acc_ref[...] += jnp.dot(h_ref[...], d1_ref[...],
                            preferred_element_type=jnp.float32)

    @pl.when(c == NDGRID - 1)
    def _():
        def ln(x, g_r, b_r):
            mu = jnp.mean(x, axis=1, keepdims=True)
            var = jnp.mean((x - mu) ** 2, axis=1, keepdims=True)
            return g_r[...] * (x - mu) / jnp.sqrt(var + 1e-5) + b_r[...]

        h = _lrelu(ln(acc_ref[...] + db1_ref[...], dg1_ref, dbe1_ref))
        h = _lrelu(ln(jnp.dot(h, d2_ref[...], preferred_element_type=jnp.float32)
                      + db2_ref[...], dg2_ref, dbe2_ref))
        h = _lrelu(ln(jnp.dot(h, d3_ref[...], preferred_element_type=jnp.float32)
                      + db3_ref[...], dg3_ref, dbe3_ref))
        z = jnp.dot(h, fc_ref[...], preferred_element_type=jnp.float32) + fcb_ref[...]
        hyp_ref[...] = jnp.clip(jax.nn.sigmoid(z), 1e-4, 0.99)


def _hyp_coefs(hyp_ref):
    alpha = hyp_ref[:, 0:1][:, :, None] * ALPHA_MAX
    tau = hyp_ref[:, 1:2][:, :, None] * TAU_MAX
    rho = hyp_ref[:, 2:3][:, :, None] * RHO_MAX
    eta = hyp_ref[:, 3:4][:, :, None] * ETA_MAX
    return alpha, tau, rho, eta


def _update_kernel(y_ref, u_ref, dl_ref, ay_ref, atb_ref, hyp_ref, lap_ref,
                   deg_ref, yn_ref, dn_ref, un_ref, *, mg, mv):
    alpha, tau, rho, eta = _hyp_coefs(hyp_ref)
    degc = deg_ref[:, 0:1].reshape(1, PP, 1)
    y = y_ref[...]
    grad = (ay_ref[...] - atb_ref[...] + jnp.sign(y) * tau
            + u_ref[...] * degc + dl_ref[...] * rho)
    grad = jnp.clip(grad, -mg, mg)
    y_n = jnp.clip(y - alpha * grad, -mv, mv)
    l_b = jnp.broadcast_to(lap_ref[...], (B, PP, PP))
    d_n = jax.lax.dot_general(l_b, y_n, (((2,), (1,)), ((0,), (0,))),
                              preferred_element_type=jnp.float32)
    yn_ref[...] = y_n
    dn_ref[...] = d_n
    un_ref[...] = jnp.clip(u_ref[...] + d_n * eta, -mv, mv)


def _update_kernel_k0(atb_ref, hyp_ref, lap_ref, yn_ref, dn_ref, un_ref,
                      *, mg, mv):
    alpha, tau, rho, eta = _hyp_coefs(hyp_ref)
    grad = jnp.clip(-atb_ref[...], -mg, mg)
    y_n = jnp.clip(-alpha * grad, -mv, mv)
    l_b = jnp.broadcast_to(lap_ref[...], (B, PP, PP))
    d_n = jax.lax.dot_general(l_b, y_n, (((2,), (1,)), ((0,), (0,))),
                              preferred_element_type=jnp.float32)
    yn_ref[...] = y_n
    dn_ref[...] = d_n
    un_ref[...] = jnp.clip(d_n * eta, -mv, mv)


def _full(shape):
    return pl.BlockSpec(shape, lambda *_: tuple(0 for _ in shape))


def kernel(b, A, params, edge_index):
    p = params
    f32 = jnp.float32
    A0 = A[0]
    bmat = b[..., 0]

    ghat, lap, deg = pl.pallas_call(
        _edge_kernel,
        out_shape=[jax.ShapeDtypeStruct((PP, PP), f32),
                   jax.ShapeDtypeStruct((PP, PP), f32),
                   jax.ShapeDtypeStruct((PP, 8), f32)],
    )(edge_index)

    atb = pl.pallas_call(
        _atb_kernel,
        grid=(NPGRID,),
        in_specs=[pl.BlockSpec((B, PCHUNK, M), lambda j: (0, j, 0)),
                  pl.BlockSpec((PCHUNK, M, N), lambda j: (j, 0, 0))],
        out_specs=pl.BlockSpec((B, PCHUNK, N), lambda j: (0, j, 0)),
        out_shape=jax.ShapeDtypeStruct((B, P, N), f32),
    )(bmat, A0)
    atbp = jnp.pad(atb, ((0, 0), (0, PP - P), (0, 0)))

    atay = pl.pallas_call(
        _atay_kernel,
        grid=(NPGRID,),
        in_specs=[pl.BlockSpec((B, PCHUNK, N), lambda j: (0, j, 0)),
                  pl.BlockSpec((PCHUNK, M, N), lambda j: (j, 0, 0))],
        out_specs=pl.BlockSpec((B, PCHUNK, N), lambda j: (0, j, 0)),
        out_shape=jax.ShapeDtypeStruct((B, P, N), f32),
    )

    bc = jnp.concatenate([p['bc%d' % i] for i in range(1, 6)])[None, :]
    gg = jnp.concatenate([p['g%d' % i] for i in range(1, 6)])[None, :]
    be = jnp.concatenate([p['be%d' % i] for i in range(1, 6)])[None, :]
    w1a = p['W1'][:N]
    w1b = p['W1'][N:]

    gnn_w_specs = [_full((N, H)), _full((H, 2 * H)),
                   _full((2 * H, 4 * H)), _full((4 * H, 4 * H)),
                   _full((4 * H, 4 * H)),
                   _full((1, 15 * H)), _full((1, 15 * H)), _full((1, 15 * H)),
                   _full((1, 4 * H)), _full((1, 4 * H))]
    gnn_out = dict(
        out_specs=_full((B, P, 4 * H)),
        out_shape=jax.ShapeDtypeStruct((B, P, 4 * H), f32))
    gnn = pl.pallas_call(
        _gnn_kernel,
        in_specs=[_full((B, PP, N)), _full((B, PP, N)), _full((PP, PP)),
                  _full((N, H))] + gnn_w_specs, **gnn_out)
    gnn0 = pl.pallas_call(
        _gnn_kernel_k0,
        in_specs=[_full((B, PP, N)), _full((PP, PP))] + gnn_w_specs, **gnn_out)
    gnn_w = (p['W2'], p['W3'], p['W4'], p['W5'], bc, gg, be,
             p['lng'][None, :], p['lnb'][None, :])

    dec = pl.pallas_call(
        _dec_kernel,
        grid=(NDGRID,),
        in_specs=[pl.BlockSpec((B, DCHUNK), lambda c: (0, c)),
                  pl.BlockSpec((DCHUNK, 4 * H), lambda c: (c, 0)),
                  _full((1, 4 * H)), _full((1, 4 * H)), _full((1, 4 * H)),
                  _full((4 * H, 2 * H)),
                  _full((1, 2 * H)), _full((1, 2 * H)), _full((1, 2 * H)),
                  _full((2 * H, H)),
                  _full((1, H)), _full((1, H)), _full((1, H)),
                  _full((H, 4)), _full((1, 4))],
        out_specs=_full((B, 4)),
        out_shape=jax.ShapeDtypeStruct((B, 4), f32),
        scratch_shapes=[pltpu.VMEM((B, 4 * H), f32)],
    )
    dec_args = (p['db1'][None, :], p['dg1'][None, :], p['dbe1'][None, :],
                p['D2'], p['db2'][None, :], p['dg2'][None, :], p['dbe2'][None, :],
                p['D3'], p['db3'][None, :], p['dg3'][None, :], p['dbe3'][None, :],
                p['FC'], p['fcb'][None, :])

    st_shape = [jax.ShapeDtypeStruct((B, PP, N), f32)] * 3

    def make_update(k):
        mg = max(30.0, 100.0 - k)
        mv = max(10.0, 200.0 - k * 3)
        if k == 0:
            return pl.pallas_call(
                functools.partial(_update_kernel_k0, mg=mg, mv=mv),
                in_specs=[_full((B, PP, N)), _full((B, 4)), _full((PP, PP))],
                out_specs=[_full((B, PP, N))] * 3, out_shape=st_shape)
        return pl.pallas_call(
            functools.partial(_update_kernel, mg=mg, mv=mv),
            in_specs=[_full((B, PP, N))] * 5
            + [_full((B, 4)), _full((PP, PP)), _full((PP, 8))],
            out_specs=[_full((B, PP, N))] * 3, out_shape=st_shape)

    ys = []
    y_k = u_k = dl_k = None
    for k in range(K):
        if k == 0:
            hnodes = gnn0(atbp, ghat, w1b, *gnn_w)
        else:
            ayp = jnp.pad(atay(y_k, A0), ((0, 0), (0, PP - P), (0, 0)))
            hnodes = gnn(ayp, atbp, ghat, w1a, w1b, *gnn_w)
        hyp = dec(hnodes.reshape(B, P * 4 * H), p['D1'], *dec_args)
        if k == 0:
            y_k, dl_k, u_k = make_update(0)(atbp, hyp, lap)
        else:
            y_k, dl_k, u_k = make_update(k)(y_k, u_k, dl_k, ayp, atbp, hyp,
                                            lap, deg)
        ys.append(y_k[:, :P, :])
    return jnp.stack(ys)[..., None]


# speedup vs baseline: 7.1496x; 1.1294x over previous
import functools

import jax
import jax.numpy as jnp
from jax.experimental import pallas as pl
from jax.experimental.pallas import tpu as pltpu

P = 50
PP = 64
M = 256
N = 512
H = 128
B = 16
K = 3
E = 400
ALPHA_MAX = 0.01
TAU_MAX = 0.01
RHO_MAX = 0.1
ETA_MAX = 0.01

PCHUNK = 8
NPGRID = (P + PCHUNK - 1) // PCHUNK
DCHUNK = 3200
NDGRID = (P * 4 * H) // DCHUNK


def _lrelu(x):
    return jnp.where(x >= 0, x, 0.01 * x)


def _edge_kernel(ei_ref, ghat_ref, lap_ref, deg_ref):
    src = ei_ref[0:1, :]
    dst = ei_ref[1:2, :]
    rows = jax.lax.broadcasted_iota(jnp.int32, (PP, E), 0)
    s_oh = (rows == src).astype(jnp.float32)
    d_oh = (rows == dst).astype(jnp.float32)
    deg = jnp.sum(s_oh, axis=1, keepdims=True)
    dself = deg + 1.0
    ds_src = jnp.sum(s_oh * dself, axis=0, keepdims=True)
    ds_dst = jnp.sum(d_oh * dself, axis=0, keepdims=True)
    norm = jax.lax.rsqrt(ds_src * ds_dst)
    ghat = jax.lax.dot_general(
        d_oh * norm, s_oh, (((1,), (1,)), ((), ())),
        preferred_element_type=jnp.float32)
    ri = jax.lax.broadcasted_iota(jnp.int32, (PP, PP), 0)
    ci = jax.lax.broadcasted_iota(jnp.int32, (PP, PP), 1)
    eye = ((ri == ci) & (ri < P)).astype(jnp.float32)
    ghat_ref[...] = ghat + eye / dself
    dif = s_oh - d_oh
    lap_ref[...] = jax.lax.dot_general(
        dif, dif, (((1,), (1,)), ((), ())),
        preferred_element_type=jnp.float32)
    deg_ref[...] = jnp.broadcast_to(deg, (PP, 8))


def _atb_kernel(b_ref, a_ref, out_ref):
    for i in range(PCHUNK):
        out_ref[:, i, :] = jnp.dot(b_ref[:, i, :], a_ref[i],
                                   preferred_element_type=jnp.float32)


def _atay_kernel(y_ref, a_ref, out_ref):
    for i in range(PCHUNK):
        a = a_ref[i]
        u = jax.lax.dot_general(y_ref[:, i, :], a, (((1,), (1,)), ((), ())),
                                preferred_element_type=jnp.float32)
        out_ref[:, i, :] = jnp.dot(u, a, preferred_element_type=jnp.float32)


def _gnn_stack(ay_ref, atb_ref, g_ref,
               w1a_ref, w1b_ref, w2_ref, w3_ref, w4_ref, w5_ref,
               bc_ref, gg_ref, be_ref, lng_ref, lnb_ref, out_ref):
    g_b = jnp.broadcast_to(g_ref[...], (B, PP, PP))
    mask = (jax.lax.broadcasted_iota(jnp.int32, (1, PP, 1), 1)
            < P).astype(jnp.float32)

    def layer(xw, lo, fo):
        x3 = xw.reshape(B, PP, fo)
        agg = jax.lax.dot_general(
            g_b, x3, (((2,), (1,)), ((0,), (0,))),
            preferred_element_type=jnp.float32)
        h = _lrelu(agg + bc_ref[0:1, lo:lo + fo].reshape(1, 1, fo))
        mu = jnp.sum(h * mask, axis=1, keepdims=True) / P
        var = jnp.sum(((h - mu) ** 2) * mask, axis=1, keepdims=True) / P
        h = (gg_ref[0:1, lo:lo + fo].reshape(1, 1, fo) * (h - mu)
             / jnp.sqrt(var + 1e-5)
             + be_ref[0:1, lo:lo + fo].reshape(1, 1, fo))
        return h.reshape(B * PP, fo)

    x2b = atb_ref[...].reshape(B * PP, N)
    xw = jnp.dot(x2b, w1b_ref[...], preferred_element_type=jnp.float32)
    if ay_ref is not None:
        x2a = ay_ref[...].reshape(B * PP, N)
        xw = xw + jnp.dot(x2a, w1a_ref[...], preferred_element_type=jnp.float32)
    h = layer(xw, 0, H)
    h = layer(jnp.dot(h, w2_ref[...], preferred_element_type=jnp.float32),
              H, 2 * H)
    h = layer(jnp.dot(h, w3_ref[...], preferred_element_type=jnp.float32),
              3 * H, 4 * H)
    h = layer(jnp.dot(h, w4_ref[...], preferred_element_type=jnp.float32),
              7 * H, 4 * H)
    h = layer(jnp.dot(h, w5_ref[...], preferred_element_type=jnp.float32),
              11 * H, 4 * H)
    h3 = h.reshape(B, PP, 4 * H)
    mu = jnp.mean(h3, axis=2, keepdims=True)
    var = jnp.mean((h3 - mu) ** 2, axis=2, keepdims=True)
    hln = (lng_ref[...].reshape(1, 1, 4 * H) * (h3 - mu)
           / jnp.sqrt(var + 1e-5) + lnb_ref[...].reshape(1, 1, 4 * H))
    out_ref[...] = hln


def _dec_tail(hsc_ref, dsc_ref, db1_ref, dg1_ref, dbe1_ref,
              d2_ref, db2_ref, dg2_ref, dbe2_ref,
              d3_ref, db3_ref, dg3_ref, dbe3_ref,
              fc_ref, fcb_ref, hyp_ref):
    acc = jnp.zeros((B, 4 * H), jnp.float32)
    for q in range(P):
        acc = acc + jnp.dot(hsc_ref[:, q, :], dsc_ref[q],
                            preferred_element_type=jnp.float32)

    def ln(x, g_r, b_r):
        mu = jnp.mean(x, axis=1, keepdims=True)
        var = jnp.mean((x - mu) ** 2, axis=1, keepdims=True)
        return g_r[...] * (x - mu) / jnp.sqrt(var + 1e-5) + b_r[...]

    h = _lrelu(ln(acc + db1_ref[...], dg1_ref, dbe1_ref))
    h = _lrelu(ln(jnp.dot(h, d2_ref[...], preferred_element_type=jnp.float32)
                  + db2_ref[...], dg2_ref, dbe2_ref))
    h = _lrelu(ln(jnp.dot(h, d3_ref[...], preferred_element_type=jnp.float32)
                  + db3_ref[...], dg3_ref, dbe3_ref))
    z = jnp.dot(h, fc_ref[...], preferred_element_type=jnp.float32) + fcb_ref[...]
    hyp_ref[...] = jnp.clip(jax.nn.sigmoid(z), 1e-4, 0.99)


def _gnndec_kernel(ay_ref, atb_ref, g_ref,
                   w1a_ref, w1b_ref, w2_ref, w3_ref, w4_ref, w5_ref,
                   bc_ref, gg_ref, be_ref, lng_ref, lnb_ref,
                   d1_ref, db1_ref, dg1_ref, dbe1_ref,
                   d2_ref, db2_ref, dg2_ref, dbe2_ref,
                   d3_ref, db3_ref, dg3_ref, dbe3_ref,
                   fc_ref, fcb_ref, hyp_ref, dsc_ref, hsc_ref, sem):
    s = pl.program_id(0)

    @pl.when(s == 0)
    def _():
        pltpu.make_async_copy(d1_ref, dsc_ref, sem).start()
        _gnn_stack(ay_ref, atb_ref, g_ref, w1a_ref, w1b_ref, w2_ref, w3_ref,
                   w4_ref, w5_ref, bc_ref, gg_ref, be_ref, lng_ref, lnb_ref,
                   hsc_ref)

    @pl.when(s == 1)
    def _():
        pltpu.make_async_copy(d1_ref, dsc_ref, sem).wait()
        _dec_tail(hsc_ref, dsc_ref, db1_ref, dg1_ref, dbe1_ref,
                  d2_ref, db2_ref, dg2_ref, dbe2_ref,
                  d3_ref, db3_ref, dg3_ref, dbe3_ref,
                  fc_ref, fcb_ref, hyp_ref)


def _gnndec_kernel_k0(atb_ref, g_ref, w1b_ref, w2_ref, w3_ref, w4_ref, w5_ref,
                      bc_ref, gg_ref, be_ref, lng_ref, lnb_ref,
                      d1_ref, db1_ref, dg1_ref, dbe1_ref,
                      d2_ref, db2_ref, dg2_ref, dbe2_ref,
                      d3_ref, db3_ref, dg3_ref, dbe3_ref,
                      fc_ref, fcb_ref, hyp_ref, dsc_ref, hsc_ref, sem):
    _gnndec_kernel(None, atb_ref, g_ref, None, w1b_ref, w2_ref, w3_ref,
                   w4_ref, w5_ref, bc_ref, gg_ref, be_ref, lng_ref, lnb_ref,
                   d1_ref, db1_ref, dg1_ref, dbe1_ref,
                   d2_ref, db2_ref, dg2_ref, dbe2_ref,
                   d3_ref, db3_ref, dg3_ref, dbe3_ref,
                   fc_ref, fcb_ref, hyp_ref, dsc_ref, hsc_ref, sem)


def _hyp_coefs(hyp_ref):
    alpha = hyp_ref[:, 0:1][:, :, None] * ALPHA_MAX
    tau = hyp_ref[:, 1:2][:, :, None] * TAU_MAX
    rho = hyp_ref[:, 2:3][:, :, None] * RHO_MAX
    eta = hyp_ref[:, 3:4][:, :, None] * ETA_MAX
    return alpha, tau, rho, eta


def _update_kernel(y_ref, u_ref, dl_ref, ay_ref, atb_ref, hyp_ref, lap_ref,
                   deg_ref, yn_ref, dn_ref, un_ref, *, mg, mv):
    alpha, tau, rho, eta = _hyp_coefs(hyp_ref)
    degc = deg_ref[:, 0:1].reshape(1, PP, 1)
    y = y_ref[...]
    grad = (ay_ref[...] - atb_ref[...] + jnp.sign(y) * tau
            + u_ref[...] * degc + dl_ref[...] * rho)
    grad = jnp.clip(grad, -mg, mg)
    y_n = jnp.clip(y - alpha * grad, -mv, mv)
    l_b = jnp.broadcast_to(lap_ref[...], (B, PP, PP))
    d_n = jax.lax.dot_general(l_b, y_n, (((2,), (1,)), ((0,), (0,))),
                              preferred_element_type=jnp.float32)
    yn_ref[...] = y_n
    dn_ref[...] = d_n
    un_ref[...] = jnp.clip(u_ref[...] + d_n * eta, -mv, mv)


def _update_kernel_k0(atb_ref, hyp_ref, lap_ref, yn_ref, dn_ref, un_ref,
                      *, mg, mv):
    alpha, tau, rho, eta = _hyp_coefs(hyp_ref)
    grad = jnp.clip(-atb_ref[...], -mg, mg)
    y_n = jnp.clip(-alpha * grad, -mv, mv)
    l_b = jnp.broadcast_to(lap_ref[...], (B, PP, PP))
    d_n = jax.lax.dot_general(l_b, y_n, (((2,), (1,)), ((0,), (0,))),
                              preferred_element_type=jnp.float32)
    yn_ref[...] = y_n
    dn_ref[...] = d_n
    un_ref[...] = jnp.clip(d_n * eta, -mv, mv)


def _full(shape):
    return pl.BlockSpec(shape, lambda *_: tuple(0 for _ in shape))


def kernel(b, A, params, edge_index):
    p = params
    f32 = jnp.float32
    A0 = A[0]
    bmat = b[..., 0]

    ghat, lap, deg = pl.pallas_call(
        _edge_kernel,
        out_shape=[jax.ShapeDtypeStruct((PP, PP), f32),
                   jax.ShapeDtypeStruct((PP, PP), f32),
                   jax.ShapeDtypeStruct((PP, 8), f32)],
    )(edge_index)

    atb = pl.pallas_call(
        _atb_kernel,
        grid=(NPGRID,),
        in_specs=[pl.BlockSpec((B, PCHUNK, M), lambda j: (0, j, 0)),
                  pl.BlockSpec((PCHUNK, M, N), lambda j: (j, 0, 0))],
        out_specs=pl.BlockSpec((B, PCHUNK, N), lambda j: (0, j, 0)),
        out_shape=jax.ShapeDtypeStruct((B, P, N), f32),
    )(bmat, A0)
    atbp = jnp.pad(atb, ((0, 0), (0, PP - P), (0, 0)))

    atay = pl.pallas_call(
        _atay_kernel,
        grid=(NPGRID,),
        in_specs=[pl.BlockSpec((B, PCHUNK, N), lambda j: (0, j, 0)),
                  pl.BlockSpec((PCHUNK, M, N), lambda j: (j, 0, 0))],
        out_specs=pl.BlockSpec((B, PCHUNK, N), lambda j: (0, j, 0)),
        out_shape=jax.ShapeDtypeStruct((B, P, N), f32),
    )

    bc = jnp.concatenate([p['bc%d' % i] for i in range(1, 6)])[None, :]
    gg = jnp.concatenate([p['g%d' % i] for i in range(1, 6)])[None, :]
    be = jnp.concatenate([p['be%d' % i] for i in range(1, 6)])[None, :]
    w1a = p['W1'][:N]
    w1b = p['W1'][N:]

    gnn_w_specs = [_full((N, H)), _full((H, 2 * H)),
                   _full((2 * H, 4 * H)), _full((4 * H, 4 * H)),
                   _full((4 * H, 4 * H)),
                   _full((1, 15 * H)), _full((1, 15 * H)), _full((1, 15 * H)),
                   _full((1, 4 * H)), _full((1, 4 * H))]
    dec_specs = [pl.BlockSpec(memory_space=pl.ANY),
                 _full((1, 4 * H)), _full((1, 4 * H)), _full((1, 4 * H)),
                 _full((4 * H, 2 * H)),
                 _full((1, 2 * H)), _full((1, 2 * H)), _full((1, 2 * H)),
                 _full((2 * H, H)),
                 _full((1, H)), _full((1, H)), _full((1, H)),
                 _full((H, 4)), _full((1, 4))]
    gnndec_kw = dict(
        grid=(2,),
        out_specs=_full((B, 4)),
        out_shape=jax.ShapeDtypeStruct((B, 4), f32),
        scratch_shapes=[pltpu.VMEM((P, 4 * H, 4 * H), f32),
                        pltpu.VMEM((B, PP, 4 * H), f32),
                        pltpu.SemaphoreType.DMA],
        compiler_params=pltpu.CompilerParams(
            vmem_limit_bytes=110 * 1024 * 1024),
    )
    gnndec = pl.pallas_call(
        _gnndec_kernel,
        in_specs=[_full((B, PP, N)), _full((B, PP, N)), _full((PP, PP)),
                  _full((N, H))] + gnn_w_specs + dec_specs, **gnndec_kw)
    gnndec0 = pl.pallas_call(
        _gnndec_kernel_k0,
        in_specs=[_full((B, PP, N)), _full((PP, PP))] + gnn_w_specs
        + dec_specs, **gnndec_kw)
    gnn_w = (p['W2'], p['W3'], p['W4'], p['W5'], bc, gg, be,
             p['lng'][None, :], p['lnb'][None, :])
    d1_3d = p['D1'].reshape(P, 4 * H, 4 * H)
    dec_args = (d1_3d,
                p['db1'][None, :], p['dg1'][None, :], p['dbe1'][None, :],
                p['D2'], p['db2'][None, :], p['dg2'][None, :], p['dbe2'][None, :],
                p['D3'], p['db3'][None, :], p['dg3'][None, :], p['dbe3'][None, :],
                p['FC'], p['fcb'][None, :])

    st_shape = [jax.ShapeDtypeStruct((B, PP, N), f32)] * 3

    def make_update(k):
        mg = max(30.0, 100.0 - k)
        mv = max(10.0, 200.0 - k * 3)
        if k == 0:
            return pl.pallas_call(
                functools.partial(_update_kernel_k0, mg=mg, mv=mv),
                in_specs=[_full((B, PP, N)), _full((B, 4)), _full((PP, PP))],
                out_specs=[_full((B, PP, N))] * 3, out_shape=st_shape)
        return pl.pallas_call(
            functools.partial(_update_kernel, mg=mg, mv=mv),
            in_specs=[_full((B, PP, N))] * 5
            + [_full((B, 4)), _full((PP, PP)), _full((PP, 8))],
            out_specs=[_full((B, PP, N))] * 3, out_shape=st_shape)

    ys = []
    y_k = u_k = dl_k = None
    for k in range(K):
        if k == 0:
            hyp = gnndec0(atbp, ghat, w1b, *gnn_w, *dec_args)
            y_k, dl_k, u_k = make_update(0)(atbp, hyp, lap)
        else:
            ayp = jnp.pad(atay(y_k, A0), ((0, 0), (0, PP - P), (0, 0)))
            hyp = gnndec(ayp, atbp, ghat, w1a, w1b, *gnn_w, *dec_args)
            y_k, dl_k, u_k = make_update(k)(y_k, u_k, dl_k, ayp, atbp, hyp,
                                            lap, deg)
        ys.append(y_k[:, :P, :])
    return jnp.stack(ys)[..., None]


# speedup vs baseline: 8.8517x; 1.2381x over previous
import jax
import jax.numpy as jnp
from jax.experimental import pallas as pl
from jax.experimental.pallas import tpu as pltpu

P = 50
PP = 64
M = 256
N = 512
H = 128
B = 16
K = 3
E = 400
ALPHA_MAX = 0.01
TAU_MAX = 0.01
RHO_MAX = 0.1
ETA_MAX = 0.01


def _lrelu(x):
    return jnp.where(x >= 0, x, 0.01 * x)


def _edge_compute(ei_ref, ghat_s, lap_s, deg_s):
    src = ei_ref[0:1, :]
    dst = ei_ref[1:2, :]
    rows = jax.lax.broadcasted_iota(jnp.int32, (PP, E), 0)
    s_oh = (rows == src).astype(jnp.float32)
    d_oh = (rows == dst).astype(jnp.float32)
    deg = jnp.sum(s_oh, axis=1, keepdims=True)
    dself = deg + 1.0
    ds_src = jnp.sum(s_oh * dself, axis=0, keepdims=True)
    ds_dst = jnp.sum(d_oh * dself, axis=0, keepdims=True)
    norm = jax.lax.rsqrt(ds_src * ds_dst)
    ghat = jax.lax.dot_general(
        d_oh * norm, s_oh, (((1,), (1,)), ((), ())),
        preferred_element_type=jnp.float32)
    ri = jax.lax.broadcasted_iota(jnp.int32, (PP, PP), 0)
    ci = jax.lax.broadcasted_iota(jnp.int32, (PP, PP), 1)
    eye = ((ri == ci) & (ri < P)).astype(jnp.float32)
    ghat_s[...] = ghat + eye / dself
    dif = s_oh - d_oh
    lap_s[...] = jax.lax.dot_general(
        dif, dif, (((1,), (1,)), ((), ())),
        preferred_element_type=jnp.float32)
    deg_s[...] = jnp.broadcast_to(deg, (PP, 8))


def _gnn_compute(ay_s, atb_s, ghat_s, w1a_ref, w1b_ref, w2_ref, w3_ref,
                 w4_ref, w5_ref, bc_ref, gg_ref, be_ref, lng_ref, lnb_ref,
                 h_s):
    g_b = jnp.broadcast_to(ghat_s[...], (B, PP, PP))
    mask = (jax.lax.broadcasted_iota(jnp.int32, (1, PP, 1), 1)
            < P).astype(jnp.float32)

    def layer(xw, lo, fo):
        x3 = xw.reshape(B, PP, fo)
        agg = jax.lax.dot_general(
            g_b, x3, (((2,), (1,)), ((0,), (0,))),
            preferred_element_type=jnp.float32)
        h = _lrelu(agg + bc_ref[0:1, lo:lo + fo].reshape(1, 1, fo))
        mu = jnp.sum(h * mask, axis=1, keepdims=True) / P
        var = jnp.sum(((h - mu) ** 2) * mask, axis=1, keepdims=True) / P
        h = (gg_ref[0:1, lo:lo + fo].reshape(1, 1, fo) * (h - mu)
             / jnp.sqrt(var + 1e-5)
             + be_ref[0:1, lo:lo + fo].reshape(1, 1, fo))
        return h.reshape(B * PP, fo)

    xw = (jnp.dot(ay_s[...].reshape(B * PP, N), w1a_ref[...],
                  preferred_element_type=jnp.float32)
          + jnp.dot(atb_s[...].reshape(B * PP, N), w1b_ref[...],
                    preferred_element_type=jnp.float32))
    h = layer(xw, 0, H)
    h = layer(jnp.dot(h, w2_ref[...], preferred_element_type=jnp.float32),
              H, 2 * H)
    h = layer(jnp.dot(h, w3_ref[...], preferred_element_type=jnp.float32),
              3 * H, 4 * H)
    h = layer(jnp.dot(h, w4_ref[...], preferred_element_type=jnp.float32),
              7 * H, 4 * H)
    h = layer(jnp.dot(h, w5_ref[...], preferred_element_type=jnp.float32),
              11 * H, 4 * H)
    h3 = h.reshape(B, PP, 4 * H)
    mu = jnp.mean(h3, axis=2, keepdims=True)
    var = jnp.mean((h3 - mu) ** 2, axis=2, keepdims=True)
    h_s[...] = (lng_ref[...].reshape(1, 1, 4 * H) * (h3 - mu)
                / jnp.sqrt(var + 1e-5) + lnb_ref[...].reshape(1, 1, 4 * H))


DCH = 4
DSTART = list(range(0, P, DCH))
DSIZE = [min(DCH, P - s) for s in DSTART]
NDC = len(DSTART)
NRING = 3


def _d1_copy(d1_any, rings, sems, c):
    slot = c % NRING
    return pltpu.make_async_copy(
        d1_any.at[DSTART[c]:DSTART[c] + DSIZE[c]],
        rings[slot].at[0:DSIZE[c]], sems[slot])


def _dec_compute(h_s, d1_any, rings, sems, fwd,
                 db1_ref, dg1_ref, dbe1_ref,
                 d2_ref, db2_ref, dg2_ref, dbe2_ref,
                 d3_ref, db3_ref, dg3_ref, dbe3_ref,
                 fc_ref, fcb_ref, hyp_s):
    acc = jnp.zeros((B, 4 * H), jnp.float32)
    order = list(range(NDC)) if fwd else list(range(NDC - 1, -1, -1))
    for pos, c in enumerate(order):
        slot = c % NRING
        if pos >= NRING:
            _d1_copy(d1_any, rings, sems, c).wait()
        for i in range(DSIZE[c]):
            acc = acc + jnp.dot(h_s[:, DSTART[c] + i, :], rings[slot][i],
                                preferred_element_type=jnp.float32)
        if pos + NRING < NDC:
            _d1_copy(d1_any, rings, sems, order[pos + NRING]).start()

    def ln(x, g_r, b_r):
        mu = jnp.mean(x, axis=1, keepdims=True)
        var = jnp.mean((x - mu) ** 2, axis=1, keepdims=True)
        return g_r[...] * (x - mu) / jnp.sqrt(var + 1e-5) + b_r[...]

    h = _lrelu(ln(acc + db1_ref[...], dg1_ref, dbe1_ref))
    h = _lrelu(ln(jnp.dot(h, d2_ref[...], preferred_element_type=jnp.float32)
                  + db2_ref[...], dg2_ref, dbe2_ref))
    h = _lrelu(ln(jnp.dot(h, d3_ref[...], preferred_element_type=jnp.float32)
                  + db3_ref[...], dg3_ref, dbe3_ref))
    z = jnp.dot(h, fc_ref[...], preferred_element_type=jnp.float32) + fcb_ref[...]
    hyp_s[...] = jnp.clip(jax.nn.sigmoid(z), 1e-4, 0.99)


def _mega_kernel(ei_ref, b_ref, a_any, d1_any,
                 w1a_ref, w1b_ref, w2_ref, w3_ref, w4_ref, w5_ref,
                 bc_ref, gg_ref, be_ref, lng_ref, lnb_ref,
                 db1_ref, dg1_ref, dbe1_ref,
                 d2_ref, db2_ref, dg2_ref, dbe2_ref,
                 d3_ref, db3_ref, dg3_ref, dbe3_ref,
                 fc_ref, fcb_ref,
                 out_ref,
                 asc, ring0, ring1, ring2, ghat_s, lap_s, deg_s, atb_s, ay_s,
                 y_s, u_s, dl_s, h_s, hyp_s, sem_a, sem_d0, sem_d1, sem_d2):
    k = pl.program_id(0)
    s = pl.program_id(1)
    rings = [ring0, ring1, ring2]
    sems = [sem_d0, sem_d1, sem_d2]

    @pl.when((k == 0) & (s == 0))
    def _():
        pltpu.make_async_copy(a_any, asc, sem_a).start()
        for c in range(NRING):
            pltpu.make_async_copy(d1_any.at[DSTART[c]:DSTART[c] + DSIZE[c]],
                                  rings[c].at[0:DSIZE[c]], sems[c]).start()
        _edge_compute(ei_ref, ghat_s, lap_s, deg_s)
        zero = jnp.zeros((B, PP, N), jnp.float32)
        ay_s[...] = zero
        y_s[...] = zero
        u_s[...] = zero
        dl_s[...] = zero

    @pl.when((k == 0) & (s == 1))
    def _():
        pltpu.make_async_copy(a_any, asc, sem_a).wait()
        for q in range(P):
            atb_s[:, q, :] = jnp.dot(b_ref[:, q, :], asc[q],
                                     preferred_element_type=jnp.float32)
        atb_s[:, P:, :] = jnp.zeros((B, PP - P, N), jnp.float32)

    @pl.when((k > 0) & (s == 1))
    def _():
        for q in range(P):
            a = asc[q]
            u = jax.lax.dot_general(
                y_s[:, q, :], a, (((1,), (1,)), ((), ())),
                preferred_element_type=jnp.float32)
            ay_s[:, q, :] = jnp.dot(u, a, preferred_element_type=jnp.float32)

    @pl.when((k == 0) & (s == 2))
    def _():
        for c in range(NRING):
            _d1_copy(d1_any, rings, sems, c).wait()

    @pl.when(s == 2)
    def _():
        _gnn_compute(ay_s, atb_s, ghat_s, w1a_ref, w1b_ref, w2_ref, w3_ref,
                     w4_ref, w5_ref, bc_ref, gg_ref, be_ref, lng_ref,
                     lnb_ref, h_s)

    @pl.when(s == 3)
    def _():
        for fwd in (True, False):
            @pl.when((k % 2) == (0 if fwd else 1))
            def _():
                _dec_compute(h_s, d1_any, rings, sems, fwd,
                             db1_ref, dg1_ref, dbe1_ref,
                             d2_ref, db2_ref, dg2_ref, dbe2_ref,
                             d3_ref, db3_ref, dg3_ref, dbe3_ref,
                             fc_ref, fcb_ref, hyp_s)

    @pl.when(s == 4)
    def _():
        kf = k.astype(jnp.float32)
        mg = 100.0 - kf
        mv = 200.0 - 3.0 * kf
        alpha = hyp_s[:, 0:1][:, :, None] * ALPHA_MAX
        tau = hyp_s[:, 1:2][:, :, None] * TAU_MAX
        rho = hyp_s[:, 2:3][:, :, None] * RHO_MAX
        eta = hyp_s[:, 3:4][:, :, None] * ETA_MAX
        degc = deg_s[:, 0:1].reshape(1, PP, 1)
        y = y_s[...]
        grad = (ay_s[...] - atb_s[...] + jnp.sign(y) * tau
                + u_s[...] * degc + dl_s[...] * rho)
        grad = jnp.clip(grad, -mg, mg)
        y_n = jnp.clip(y - alpha * grad, -mv, mv)
        l_b = jnp.broadcast_to(lap_s[...], (B, PP, PP))
        d_n = jax.lax.dot_general(l_b, y_n, (((2,), (1,)), ((0,), (0,))),
                                  preferred_element_type=jnp.float32)
        y_s[...] = y_n
        dl_s[...] = d_n
        u_s[...] = jnp.clip(u_s[...] + d_n * eta, -mv, mv)
        out_ref[0] = y_n[:, :P, :]


def _full(shape):
    return pl.BlockSpec(shape, lambda *_: tuple(0 for _ in shape))


def kernel(b, A, params, edge_index):
    p = params
    f32 = jnp.float32
    A0 = A[0]
    bmat = b[..., 0]
    d1_3d = p['D1'].reshape(P, 4 * H, 4 * H)

    bc = jnp.concatenate([p['bc%d' % i] for i in range(1, 6)])[None, :]
    gg = jnp.concatenate([p['g%d' % i] for i in range(1, 6)])[None, :]
    be = jnp.concatenate([p['be%d' % i] for i in range(1, 6)])[None, :]
    w1a = p['W1'][:N]
    w1b = p['W1'][N:]

    out = pl.pallas_call(
        _mega_kernel,
        grid=(K, 5),
        in_specs=[_full((2, E)), _full((B, P, M)),
                  pl.BlockSpec(memory_space=pl.ANY),
                  pl.BlockSpec(memory_space=pl.ANY),
                  _full((N, H)), _full((N, H)), _full((H, 2 * H)),
                  _full((2 * H, 4 * H)), _full((4 * H, 4 * H)),
                  _full((4 * H, 4 * H)),
                  _full((1, 15 * H)), _full((1, 15 * H)), _full((1, 15 * H)),
                  _full((1, 4 * H)), _full((1, 4 * H)),
                  _full((1, 4 * H)), _full((1, 4 * H)), _full((1, 4 * H)),
                  _full((4 * H, 2 * H)),
                  _full((1, 2 * H)), _full((1, 2 * H)), _full((1, 2 * H)),
                  _full((2 * H, H)),
                  _full((1, H)), _full((1, H)), _full((1, H)),
                  _full((H, 4)), _full((1, 4))],
        out_specs=pl.BlockSpec((1, B, P, N), lambda k, s: (k, 0, 0, 0)),
        out_shape=jax.ShapeDtypeStruct((K, B, P, N), f32),
        scratch_shapes=[pltpu.VMEM((P, M, N), f32),
                        pltpu.VMEM((DCH, 4 * H, 4 * H), f32),
                        pltpu.VMEM((DCH, 4 * H, 4 * H), f32),
                        pltpu.VMEM((DCH, 4 * H, 4 * H), f32),
                        pltpu.VMEM((PP, PP), f32),
                        pltpu.VMEM((PP, PP), f32),
                        pltpu.VMEM((PP, 8), f32),
                        pltpu.VMEM((B, PP, N), f32),
                        pltpu.VMEM((B, PP, N), f32),
                        pltpu.VMEM((B, PP, N), f32),
                        pltpu.VMEM((B, PP, N), f32),
                        pltpu.VMEM((B, PP, N), f32),
                        pltpu.VMEM((B, PP, 4 * H), f32),
                        pltpu.VMEM((B, 4), f32),
                        pltpu.SemaphoreType.DMA,
                        pltpu.SemaphoreType.DMA,
                        pltpu.SemaphoreType.DMA,
                        pltpu.SemaphoreType.DMA],
        compiler_params=pltpu.CompilerParams(
            vmem_limit_bytes=128 * 1024 * 1024),
    )(edge_index, bmat, A0, d1_3d, w1a, w1b, p['W2'], p['W3'], p['W4'],
      p['W5'], bc, gg, be, p['lng'][None, :], p['lnb'][None, :],
      p['db1'][None, :], p['dg1'][None, :], p['dbe1'][None, :],
      p['D2'], p['db2'][None, :], p['dg2'][None, :], p['dbe2'][None, :],
      p['D3'], p['db3'][None, :], p['dg3'][None, :], p['dbe3'][None, :],
      p['FC'], p['fcb'][None, :])
    return out[..., None]


# speedup vs baseline: 9.0201x; 1.0190x over previous
import jax
import jax.numpy as jnp
from jax.experimental import pallas as pl
from jax.experimental.pallas import tpu as pltpu

P = 50
PP = 56
M = 256
N = 512
H = 128
B = 16
K = 3
E = 400
ALPHA_MAX = 0.01
TAU_MAX = 0.01
RHO_MAX = 0.1
ETA_MAX = 0.01


def _lrelu(x):
    return jnp.where(x >= 0, x, 0.01 * x)


def _edge_compute(ei_ref, ghat_s, lap_s, deg_s):
    src = ei_ref[0:1, :]
    dst = ei_ref[1:2, :]
    rows = jax.lax.broadcasted_iota(jnp.int32, (PP, E), 0)
    s_oh = (rows == src).astype(jnp.float32)
    d_oh = (rows == dst).astype(jnp.float32)
    deg = jnp.sum(s_oh, axis=1, keepdims=True)
    dself = deg + 1.0
    ds_src = jnp.sum(s_oh * dself, axis=0, keepdims=True)
    ds_dst = jnp.sum(d_oh * dself, axis=0, keepdims=True)
    norm = jax.lax.rsqrt(ds_src * ds_dst)
    ghat = jax.lax.dot_general(
        d_oh * norm, s_oh, (((1,), (1,)), ((), ())),
        preferred_element_type=jnp.float32)
    ri = jax.lax.broadcasted_iota(jnp.int32, (PP, PP), 0)
    ci = jax.lax.broadcasted_iota(jnp.int32, (PP, PP), 1)
    eye = ((ri == ci) & (ri < P)).astype(jnp.float32)
    ghat_s[...] = ghat + eye / dself
    dif = s_oh - d_oh
    lap_s[...] = jax.lax.dot_general(
        dif, dif, (((1,), (1,)), ((), ())),
        preferred_element_type=jnp.float32)
    deg_s[...] = jnp.broadcast_to(deg, (PP, 8))


def _gnn_compute(ay_s, atb_s, ghat_s, w1a_ref, w1b_ref, w2_ref, w3_ref,
                 w4_ref, w5_ref, bc_ref, gg_ref, be_ref, lng_ref, lnb_ref,
                 h_s):
    g_b = jnp.broadcast_to(ghat_s[...], (B, PP, PP))
    mask = (jax.lax.broadcasted_iota(jnp.int32, (1, PP, 1), 1)
            < P).astype(jnp.float32)

    def layer(xw, lo, fo):
        x3 = xw.reshape(B, PP, fo)
        agg = jax.lax.dot_general(
            g_b, x3, (((2,), (1,)), ((0,), (0,))),
            preferred_element_type=jnp.float32)
        h = _lrelu(agg + bc_ref[0:1, lo:lo + fo].reshape(1, 1, fo))
        mu = jnp.sum(h * mask, axis=1, keepdims=True) / P
        var = jnp.sum(((h - mu) ** 2) * mask, axis=1, keepdims=True) / P
        h = (gg_ref[0:1, lo:lo + fo].reshape(1, 1, fo) * (h - mu)
             / jnp.sqrt(var + 1e-5)
             + be_ref[0:1, lo:lo + fo].reshape(1, 1, fo))
        return h.reshape(B * PP, fo)

    xw = (jnp.dot(ay_s[...].reshape(B * PP, N), w1a_ref[...],
                  preferred_element_type=jnp.float32)
          + jnp.dot(atb_s[...].reshape(B * PP, N), w1b_ref[...],
                    preferred_element_type=jnp.float32))
    h = layer(xw, 0, H)
    h = layer(jnp.dot(h, w2_ref[...], preferred_element_type=jnp.float32),
              H, 2 * H)
    h = layer(jnp.dot(h, w3_ref[...], preferred_element_type=jnp.float32),
              3 * H, 4 * H)
    h = layer(jnp.dot(h, w4_ref[...], preferred_element_type=jnp.float32),
              7 * H, 4 * H)
    h = layer(jnp.dot(h, w5_ref[...], preferred_element_type=jnp.float32),
              11 * H, 4 * H)
    h3 = h.reshape(B, PP, 4 * H)
    mu = jnp.mean(h3, axis=2, keepdims=True)
    var = jnp.mean((h3 - mu) ** 2, axis=2, keepdims=True)
    h_s[...] = (lng_ref[...].reshape(1, 1, 4 * H) * (h3 - mu)
                / jnp.sqrt(var + 1e-5) + lnb_ref[...].reshape(1, 1, 4 * H))


DCH = 4
DSTART = list(range(0, P, DCH))
DSIZE = [min(DCH, P - s) for s in DSTART]
NDC = len(DSTART)
NRING = 3


def _d1_copy(d1_any, rings, sems, c):
    slot = c % NRING
    return pltpu.make_async_copy(
        d1_any.at[DSTART[c]:DSTART[c] + DSIZE[c]],
        rings[slot].at[0:DSIZE[c]], sems[slot])


def _dec_compute(h_s, d1_any, rings, sems, fwd,
                 db1_ref, dg1_ref, dbe1_ref,
                 d2_ref, db2_ref, dg2_ref, dbe2_ref,
                 d3_ref, db3_ref, dg3_ref, dbe3_ref,
                 fc_ref, fcb_ref, hyp_s):
    acc = jnp.zeros((B, 4 * H), jnp.float32)
    order = list(range(NDC)) if fwd else list(range(NDC - 1, -1, -1))
    for pos, c in enumerate(order):
        slot = c % NRING
        if pos >= NRING:
            _d1_copy(d1_any, rings, sems, c).wait()
        for i in range(DSIZE[c]):
            acc = acc + jnp.dot(h_s[:, DSTART[c] + i, :], rings[slot][i],
                                preferred_element_type=jnp.float32)
        if pos + NRING < NDC:
            _d1_copy(d1_any, rings, sems, order[pos + NRING]).start()

    def ln(x, g_r, b_r):
        mu = jnp.mean(x, axis=1, keepdims=True)
        var = jnp.mean((x - mu) ** 2, axis=1, keepdims=True)
        return g_r[...] * (x - mu) / jnp.sqrt(var + 1e-5) + b_r[...]

    h = _lrelu(ln(acc + db1_ref[...], dg1_ref, dbe1_ref))
    h = _lrelu(ln(jnp.dot(h, d2_ref[...], preferred_element_type=jnp.float32)
                  + db2_ref[...], dg2_ref, dbe2_ref))
    h = _lrelu(ln(jnp.dot(h, d3_ref[...], preferred_element_type=jnp.float32)
                  + db3_ref[...], dg3_ref, dbe3_ref))
    z = jnp.dot(h, fc_ref[...], preferred_element_type=jnp.float32) + fcb_ref[...]
    hyp_s[...] = jnp.clip(jax.nn.sigmoid(z), 1e-4, 0.99)


def _mega_kernel(ei_ref, b_ref, a_any, d1_any,
                 w1a_ref, w1b_ref, w2_ref, w3_ref, w4_ref, w5_ref,
                 bc_ref, gg_ref, be_ref, lng_ref, lnb_ref,
                 db1_ref, dg1_ref, dbe1_ref,
                 d2_ref, db2_ref, dg2_ref, dbe2_ref,
                 d3_ref, db3_ref, dg3_ref, dbe3_ref,
                 fc_ref, fcb_ref,
                 out_ref,
                 asc, ring0, ring1, ring2, ghat_s, lap_s, deg_s, atb_s, ay_s,
                 y_s, u_s, dl_s, h_s, hyp_s, sem_a0, sem_a1, sem_d0, sem_d1,
                 sem_d2):
    k = pl.program_id(0)
    s = pl.program_id(1)
    rings = [ring0, ring1, ring2]
    sems = [sem_d0, sem_d1, sem_d2]
    PH = P // 2

    @pl.when((k == 0) & (s == 0))
    def _():
        pltpu.make_async_copy(a_any.at[0:PH], asc.at[0:PH], sem_a0).start()
        pltpu.make_async_copy(a_any.at[PH:P], asc.at[PH:P], sem_a1).start()
        for c in range(NRING):
            pltpu.make_async_copy(d1_any.at[DSTART[c]:DSTART[c] + DSIZE[c]],
                                  rings[c].at[0:DSIZE[c]], sems[c]).start()
        _edge_compute(ei_ref, ghat_s, lap_s, deg_s)
        zero = jnp.zeros((B, PP, N), jnp.float32)
        ay_s[...] = zero
        y_s[...] = zero
        u_s[...] = zero
        dl_s[...] = zero

    @pl.when((k == 0) & (s == 1))
    def _():
        pltpu.make_async_copy(a_any.at[0:PH], asc.at[0:PH], sem_a0).wait()
        for q in range(PH):
            atb_s[:, q, :] = jnp.dot(b_ref[:, q, :], asc[q],
                                     preferred_element_type=jnp.float32)
        pltpu.make_async_copy(a_any.at[PH:P], asc.at[PH:P], sem_a1).wait()
        for q in range(PH, P):
            atb_s[:, q, :] = jnp.dot(b_ref[:, q, :], asc[q],
                                     preferred_element_type=jnp.float32)
        atb_s[:, P:, :] = jnp.zeros((B, PP - P, N), jnp.float32)

    @pl.when((k > 0) & (s == 1))
    def _():
        for q in range(P):
            a = asc[q]
            u = jax.lax.dot_general(
                y_s[:, q, :], a, (((1,), (1,)), ((), ())),
                preferred_element_type=jnp.float32)
            ay_s[:, q, :] = jnp.dot(u, a, preferred_element_type=jnp.float32)

    @pl.when((k == 0) & (s == 2))
    def _():
        for c in range(NRING):
            _d1_copy(d1_any, rings, sems, c).wait()

    @pl.when(s == 2)
    def _():
        _gnn_compute(ay_s, atb_s, ghat_s, w1a_ref, w1b_ref, w2_ref, w3_ref,
                     w4_ref, w5_ref, bc_ref, gg_ref, be_ref, lng_ref,
                     lnb_ref, h_s)

    @pl.when(s == 3)
    def _():
        for fwd in (True, False):
            @pl.when((k % 2) == (0 if fwd else 1))
            def _():
                _dec_compute(h_s, d1_any, rings, sems, fwd,
                             db1_ref, dg1_ref, dbe1_ref,
                             d2_ref, db2_ref, dg2_ref, dbe2_ref,
                             d3_ref, db3_ref, dg3_ref, dbe3_ref,
                             fc_ref, fcb_ref, hyp_s)

    @pl.when(s == 4)
    def _():
        kf = k.astype(jnp.float32)
        mg = 100.0 - kf
        mv = 200.0 - 3.0 * kf
        alpha = hyp_s[:, 0:1][:, :, None] * ALPHA_MAX
        tau = hyp_s[:, 1:2][:, :, None] * TAU_MAX
        rho = hyp_s[:, 2:3][:, :, None] * RHO_MAX
        eta = hyp_s[:, 3:4][:, :, None] * ETA_MAX
        degc = deg_s[:, 0:1].reshape(1, PP, 1)
        y = y_s[...]
        grad = (ay_s[...] - atb_s[...] + jnp.sign(y) * tau
                + u_s[...] * degc + dl_s[...] * rho)
        grad = jnp.clip(grad, -mg, mg)
        y_n = jnp.clip(y - alpha * grad, -mv, mv)
        l_b = jnp.broadcast_to(lap_s[...], (B, PP, PP))
        d_n = jax.lax.dot_general(l_b, y_n, (((2,), (1,)), ((0,), (0,))),
                                  preferred_element_type=jnp.float32)
        y_s[...] = y_n
        dl_s[...] = d_n
        u_s[...] = jnp.clip(u_s[...] + d_n * eta, -mv, mv)
        out_ref[0] = y_n[:, :P, :]


def _full(shape):
    return pl.BlockSpec(shape, lambda *_: tuple(0 for _ in shape))


def kernel(b, A, params, edge_index):
    p = params
    f32 = jnp.float32
    A0 = A[0]
    bmat = b[..., 0]
    d1_3d = p['D1'].reshape(P, 4 * H, 4 * H)

    bc = jnp.concatenate([p['bc%d' % i] for i in range(1, 6)])[None, :]
    gg = jnp.concatenate([p['g%d' % i] for i in range(1, 6)])[None, :]
    be = jnp.concatenate([p['be%d' % i] for i in range(1, 6)])[None, :]
    w1a = p['W1'][:N]
    w1b = p['W1'][N:]

    out = pl.pallas_call(
        _mega_kernel,
        grid=(K, 5),
        in_specs=[_full((2, E)), _full((B, P, M)),
                  pl.BlockSpec(memory_space=pl.ANY),
                  pl.BlockSpec(memory_space=pl.ANY),
                  _full((N, H)), _full((N, H)), _full((H, 2 * H)),
                  _full((2 * H, 4 * H)), _full((4 * H, 4 * H)),
                  _full((4 * H, 4 * H)),
                  _full((1, 15 * H)), _full((1, 15 * H)), _full((1, 15 * H)),
                  _full((1, 4 * H)), _full((1, 4 * H)),
                  _full((1, 4 * H)), _full((1, 4 * H)), _full((1, 4 * H)),
                  _full((4 * H, 2 * H)),
                  _full((1, 2 * H)), _full((1, 2 * H)), _full((1, 2 * H)),
                  _full((2 * H, H)),
                  _full((1, H)), _full((1, H)), _full((1, H)),
                  _full((H, 4)), _full((1, 4))],
        out_specs=pl.BlockSpec((1, B, P, N), lambda k, s: (k, 0, 0, 0)),
        out_shape=jax.ShapeDtypeStruct((K, B, P, N), f32),
        scratch_shapes=[pltpu.VMEM((P, M, N), f32),
                        pltpu.VMEM((DCH, 4 * H, 4 * H), f32),
                        pltpu.VMEM((DCH, 4 * H, 4 * H), f32),
                        pltpu.VMEM((DCH, 4 * H, 4 * H), f32),
                        pltpu.VMEM((PP, PP), f32),
                        pltpu.VMEM((PP, PP), f32),
                        pltpu.VMEM((PP, 8), f32),
                        pltpu.VMEM((B, PP, N), f32),
                        pltpu.VMEM((B, PP, N), f32),
                        pltpu.VMEM((B, PP, N), f32),
                        pltpu.VMEM((B, PP, N), f32),
                        pltpu.VMEM((B, PP, N), f32),
                        pltpu.VMEM((B, PP, 4 * H), f32),
                        pltpu.VMEM((B, 4), f32),
                        pltpu.SemaphoreType.DMA,
                        pltpu.SemaphoreType.DMA,
                        pltpu.SemaphoreType.DMA,
                        pltpu.SemaphoreType.DMA,
                        pltpu.SemaphoreType.DMA],
        compiler_params=pltpu.CompilerParams(
            vmem_limit_bytes=128 * 1024 * 1024),
    )(edge_index, bmat, A0, d1_3d, w1a, w1b, p['W2'], p['W3'], p['W4'],
      p['W5'], bc, gg, be, p['lng'][None, :], p['lnb'][None, :],
      p['db1'][None, :], p['dg1'][None, :], p['dbe1'][None, :],
      p['D2'], p['db2'][None, :], p['dg2'][None, :], p['dbe2'][None, :],
      p['D3'], p['db3'][None, :], p['dg3'][None, :], p['dbe3'][None, :],
      p['FC'], p['fcb'][None, :])
    return out[..., None]


# speedup vs baseline: 9.1551x; 1.0150x over previous
import jax
import jax.numpy as jnp
from jax.experimental import pallas as pl
from jax.experimental.pallas import tpu as pltpu

P = 50
PP = 56
M = 256
N = 512
H = 128
B = 16
K = 3
E = 400
ALPHA_MAX = 0.01
TAU_MAX = 0.01
RHO_MAX = 0.1
ETA_MAX = 0.01


def _lrelu(x):
    return jnp.where(x >= 0, x, 0.01 * x)


def _edge_compute(ei_ref, ghat_s, lap_s, deg_s):
    src = ei_ref[0:1, :]
    dst = ei_ref[1:2, :]
    rows = jax.lax.broadcasted_iota(jnp.int32, (PP, E), 0)
    s_oh = (rows == src).astype(jnp.float32)
    d_oh = (rows == dst).astype(jnp.float32)
    deg = jnp.sum(s_oh, axis=1, keepdims=True)
    dself = deg + 1.0
    ds_src = jnp.sum(s_oh * dself, axis=0, keepdims=True)
    ds_dst = jnp.sum(d_oh * dself, axis=0, keepdims=True)
    norm = jax.lax.rsqrt(ds_src * ds_dst)
    ghat = jax.lax.dot_general(
        d_oh * norm, s_oh, (((1,), (1,)), ((), ())),
        preferred_element_type=jnp.float32)
    ri = jax.lax.broadcasted_iota(jnp.int32, (PP, PP), 0)
    ci = jax.lax.broadcasted_iota(jnp.int32, (PP, PP), 1)
    eye = ((ri == ci) & (ri < P)).astype(jnp.float32)
    ghat_s[...] = ghat + eye / dself
    dif = s_oh - d_oh
    lap_s[...] = jax.lax.dot_general(
        dif, dif, (((1,), (1,)), ((), ())),
        preferred_element_type=jnp.float32)
    deg_s[...] = jnp.broadcast_to(deg, (PP, 8))


def _gnn_compute(ay_s, atb_s, ghat_s, w1a_ref, w1b_ref, w2_ref, w3_ref,
                 w4_ref, w5_ref, bc_ref, gg_ref, be_ref, lng_ref, lnb_ref,
                 h_s):
    g_b = jnp.broadcast_to(ghat_s[...], (B, PP, PP))
    mask = (jax.lax.broadcasted_iota(jnp.int32, (1, PP, 1), 1)
            < P).astype(jnp.float32)

    def layer(xw, lo, fo):
        x3 = xw.reshape(B, PP, fo)
        agg = jax.lax.dot_general(
            g_b, x3, (((2,), (1,)), ((0,), (0,))),
            preferred_element_type=jnp.float32)
        h = _lrelu(agg + bc_ref[0:1, lo:lo + fo].reshape(1, 1, fo))
        mu = jnp.sum(h * mask, axis=1, keepdims=True) / P
        var = jnp.sum(((h - mu) ** 2) * mask, axis=1, keepdims=True) / P
        h = (gg_ref[0:1, lo:lo + fo].reshape(1, 1, fo) * (h - mu)
             / jnp.sqrt(var + 1e-5)
             + be_ref[0:1, lo:lo + fo].reshape(1, 1, fo))
        return h.reshape(B * PP, fo)

    xw = (jnp.dot(ay_s[...].reshape(B * PP, N), w1a_ref[...],
                  preferred_element_type=jnp.float32)
          + jnp.dot(atb_s[...].reshape(B * PP, N), w1b_ref[...],
                    preferred_element_type=jnp.float32))
    h = layer(xw, 0, H)
    h = layer(jnp.dot(h, w2_ref[...], preferred_element_type=jnp.float32),
              H, 2 * H)
    h = layer(jnp.dot(h, w3_ref[...], preferred_element_type=jnp.float32),
              3 * H, 4 * H)
    h = layer(jnp.dot(h, w4_ref[...], preferred_element_type=jnp.float32),
              7 * H, 4 * H)
    h = layer(jnp.dot(h, w5_ref[...], preferred_element_type=jnp.float32),
              11 * H, 4 * H)
    h3 = h.reshape(B, PP, 4 * H)
    mu = jnp.mean(h3, axis=2, keepdims=True)
    var = jnp.mean((h3 - mu) ** 2, axis=2, keepdims=True)
    h_s[...] = (lng_ref[...].reshape(1, 1, 4 * H) * (h3 - mu)
                / jnp.sqrt(var + 1e-5) + lnb_ref[...].reshape(1, 1, 4 * H))


DCH = 4
DSTART = list(range(0, P, DCH))
DSIZE = [min(DCH, P - s) for s in DSTART]
NDC = len(DSTART)
NRING = 4


def _d1_copy(d1_any, rings, sems, c):
    slot = c % NRING
    return pltpu.make_async_copy(
        d1_any.at[DSTART[c]:DSTART[c] + DSIZE[c]],
        rings[slot].at[0:DSIZE[c]], sems[slot])


def _dec_compute(h_s, d1_any, rings, sems, fwd,
                 db1_ref, dg1_ref, dbe1_ref,
                 d2_ref, db2_ref, dg2_ref, dbe2_ref,
                 d3_ref, db3_ref, dg3_ref, dbe3_ref,
                 fc_ref, fcb_ref, hyp_s):
    acc = jnp.zeros((B, 4 * H), jnp.float32)
    order = list(range(NDC)) if fwd else list(range(NDC - 1, -1, -1))
    for pos, c in enumerate(order):
        slot = c % NRING
        if pos >= NRING:
            _d1_copy(d1_any, rings, sems, c).wait()
        for i in range(DSIZE[c]):
            acc = acc + jnp.dot(h_s[:, DSTART[c] + i, :], rings[slot][i],
                                preferred_element_type=jnp.float32)
        if pos + NRING < NDC:
            _d1_copy(d1_any, rings, sems, order[pos + NRING]).start()

    def ln(x, g_r, b_r):
        mu = jnp.mean(x, axis=1, keepdims=True)
        var = jnp.mean((x - mu) ** 2, axis=1, keepdims=True)
        return g_r[...] * (x - mu) / jnp.sqrt(var + 1e-5) + b_r[...]

    h = _lrelu(ln(acc + db1_ref[...], dg1_ref, dbe1_ref))
    h = _lrelu(ln(jnp.dot(h, d2_ref[...], preferred_element_type=jnp.float32)
                  + db2_ref[...], dg2_ref, dbe2_ref))
    h = _lrelu(ln(jnp.dot(h, d3_ref[...], preferred_element_type=jnp.float32)
                  + db3_ref[...], dg3_ref, dbe3_ref))
    z = jnp.dot(h, fc_ref[...], preferred_element_type=jnp.float32) + fcb_ref[...]
    hyp_s[...] = jnp.clip(jax.nn.sigmoid(z), 1e-4, 0.99)


def _mega_kernel(ei_ref, b_ref, a_any, d1_any,
                 w1a_ref, w1b_ref, w2_ref, w3_ref, w4_ref, w5_ref,
                 bc_ref, gg_ref, be_ref, lng_ref, lnb_ref,
                 db1_ref, dg1_ref, dbe1_ref,
                 d2_ref, db2_ref, dg2_ref, dbe2_ref,
                 d3_ref, db3_ref, dg3_ref, dbe3_ref,
                 fc_ref, fcb_ref,
                 out_ref,
                 asc, ring0, ring1, ring2, ring3, ghat_s, lap_s, deg_s, atb_s,
                 ay_s, y_s, u_s, dl_s, h_s, hyp_s, sem_a0, sem_a1, sem_d0,
                 sem_d1, sem_d2, sem_d3):
    k = pl.program_id(0)
    s = pl.program_id(1)
    rings = [ring0, ring1, ring2, ring3]
    sems = [sem_d0, sem_d1, sem_d2, sem_d3]
    PH = P // 2

    @pl.when((k == 0) & (s == 0))
    def _():
        pltpu.make_async_copy(a_any.at[0:PH], asc.at[0:PH], sem_a0).start()
        pltpu.make_async_copy(a_any.at[PH:P], asc.at[PH:P], sem_a1).start()
        for c in range(NRING):
            pltpu.make_async_copy(d1_any.at[DSTART[c]:DSTART[c] + DSIZE[c]],
                                  rings[c].at[0:DSIZE[c]], sems[c]).start()
        _edge_compute(ei_ref, ghat_s, lap_s, deg_s)
        zero = jnp.zeros((B, PP, N), jnp.float32)
        ay_s[...] = zero
        y_s[...] = zero
        u_s[...] = zero
        dl_s[...] = zero

    @pl.when((k == 0) & (s == 1))
    def _():
        pltpu.make_async_copy(a_any.at[0:PH], asc.at[0:PH], sem_a0).wait()
        for q in range(PH):
            atb_s[:, q, :] = jnp.dot(b_ref[:, q, :], asc[q],
                                     preferred_element_type=jnp.float32)
        pltpu.make_async_copy(a_any.at[PH:P], asc.at[PH:P], sem_a1).wait()
        for q in range(PH, P):
            atb_s[:, q, :] = jnp.dot(b_ref[:, q, :], asc[q],
                                     preferred_element_type=jnp.float32)
        atb_s[:, P:, :] = jnp.zeros((B, PP - P, N), jnp.float32)

    @pl.when((k > 0) & (s == 1))
    def _():
        for q in range(P):
            a = asc[q]
            u = jax.lax.dot_general(
                y_s[:, q, :], a, (((1,), (1,)), ((), ())),
                preferred_element_type=jnp.float32)
            ay_s[:, q, :] = jnp.dot(u, a, preferred_element_type=jnp.float32)

    @pl.when((k == 0) & (s == 2))
    def _():
        for c in range(NRING):
            _d1_copy(d1_any, rings, sems, c).wait()

    @pl.when(s == 2)
    def _():
        _gnn_compute(ay_s, atb_s, ghat_s, w1a_ref, w1b_ref, w2_ref, w3_ref,
                     w4_ref, w5_ref, bc_ref, gg_ref, be_ref, lng_ref,
                     lnb_ref, h_s)

    @pl.when(s == 3)
    def _():
        for fwd in (True, False):
            @pl.when((k % 2) == (0 if fwd else 1))
            def _():
                _dec_compute(h_s, d1_any, rings, sems, fwd,
                             db1_ref, dg1_ref, dbe1_ref,
                             d2_ref, db2_ref, dg2_ref, dbe2_ref,
                             d3_ref, db3_ref, dg3_ref, dbe3_ref,
                             fc_ref, fcb_ref, hyp_s)

    @pl.when(s == 4)
    def _():
        kf = k.astype(jnp.float32)
        mg = 100.0 - kf
        mv = 200.0 - 3.0 * kf
        alpha = hyp_s[:, 0:1][:, :, None] * ALPHA_MAX
        tau = hyp_s[:, 1:2][:, :, None] * TAU_MAX
        rho = hyp_s[:, 2:3][:, :, None] * RHO_MAX
        eta = hyp_s[:, 3:4][:, :, None] * ETA_MAX
        degc = deg_s[:, 0:1].reshape(1, PP, 1)
        y = y_s[...]
        grad = (ay_s[...] - atb_s[...] + jnp.sign(y) * tau
                + u_s[...] * degc + dl_s[...] * rho)
        grad = jnp.clip(grad, -mg, mg)
        y_n = jnp.clip(y - alpha * grad, -mv, mv)
        l_b = jnp.broadcast_to(lap_s[...], (B, PP, PP))
        d_n = jax.lax.dot_general(l_b, y_n, (((2,), (1,)), ((0,), (0,))),
                                  preferred_element_type=jnp.float32)
        y_s[...] = y_n
        dl_s[...] = d_n
        u_s[...] = jnp.clip(u_s[...] + d_n * eta, -mv, mv)
        out_ref[0] = y_n[:, :P, :]


def _full(shape):
    return pl.BlockSpec(shape, lambda *_: tuple(0 for _ in shape))


def kernel(b, A, params, edge_index):
    p = params
    f32 = jnp.float32
    A0 = A[0]
    bmat = b[..., 0]
    d1_3d = p['D1'].reshape(P, 4 * H, 4 * H)

    bc = jnp.concatenate([p['bc%d' % i] for i in range(1, 6)])[None, :]
    gg = jnp.concatenate([p['g%d' % i] for i in range(1, 6)])[None, :]
    be = jnp.concatenate([p['be%d' % i] for i in range(1, 6)])[None, :]
    w1a = p['W1'][:N]
    w1b = p['W1'][N:]

    out = pl.pallas_call(
        _mega_kernel,
        grid=(K, 5),
        in_specs=[_full((2, E)), _full((B, P, M)),
                  pl.BlockSpec(memory_space=pl.ANY),
                  pl.BlockSpec(memory_space=pl.ANY),
                  _full((N, H)), _full((N, H)), _full((H, 2 * H)),
                  _full((2 * H, 4 * H)), _full((4 * H, 4 * H)),
                  _full((4 * H, 4 * H)),
                  _full((1, 15 * H)), _full((1, 15 * H)), _full((1, 15 * H)),
                  _full((1, 4 * H)), _full((1, 4 * H)),
                  _full((1, 4 * H)), _full((1, 4 * H)), _full((1, 4 * H)),
                  _full((4 * H, 2 * H)),
                  _full((1, 2 * H)), _full((1, 2 * H)), _full((1, 2 * H)),
                  _full((2 * H, H)),
                  _full((1, H)), _full((1, H)), _full((1, H)),
                  _full((H, 4)), _full((1, 4))],
        out_specs=pl.BlockSpec((1, B, P, N), lambda k, s: (k, 0, 0, 0)),
        out_shape=jax.ShapeDtypeStruct((K, B, P, N), f32),
        scratch_shapes=[pltpu.VMEM((P, M, N), f32),
                        pltpu.VMEM((DCH, 4 * H, 4 * H), f32),
                        pltpu.VMEM((DCH, 4 * H, 4 * H), f32),
                        pltpu.VMEM((DCH, 4 * H, 4 * H), f32),
                        pltpu.VMEM((DCH, 4 * H, 4 * H), f32),
                        pltpu.VMEM((PP, PP), f32),
                        pltpu.VMEM((PP, PP), f32),
                        pltpu.VMEM((PP, 8), f32),
                        pltpu.VMEM((B, PP, N), f32),
                        pltpu.VMEM((B, PP, N), f32),
                        pltpu.VMEM((B, PP, N), f32),
                        pltpu.VMEM((B, PP, N), f32),
                        pltpu.VMEM((B, PP, N), f32),
                        pltpu.VMEM((B, PP, 4 * H), f32),
                        pltpu.VMEM((B, 4), f32),
                        pltpu.SemaphoreType.DMA,
                        pltpu.SemaphoreType.DMA,
                        pltpu.SemaphoreType.DMA,
                        pltpu.SemaphoreType.DMA,
                        pltpu.SemaphoreType.DMA,
                        pltpu.SemaphoreType.DMA],
        compiler_params=pltpu.CompilerParams(
            vmem_limit_bytes=128 * 1024 * 1024),
    )(edge_index, bmat, A0, d1_3d, w1a, w1b, p['W2'], p['W3'], p['W4'],
      p['W5'], bc, gg, be, p['lng'][None, :], p['lnb'][None, :],
      p['db1'][None, :], p['dg1'][None, :], p['dbe1'][None, :],
      p['D2'], p['db2'][None, :], p['dg2'][None, :], p['dbe2'][None, :],
      p['D3'], p['db3'][None, :], p['dg3'][None, :], p['dbe3'][None, :],
      p['FC'], p['fcb'][None, :])
    return out[..., None]
